# baseline (speedup 1.0000x reference)
import functools

import jax
import jax.numpy as jnp
from jax import lax
from jax.experimental import pallas as pl
from jax.experimental.pallas import tpu as pltpu
from jax.experimental.pallas import tpu_sc as plsc

ROUTEID_NUM = 100000
PAD_ID = ROUTEID_NUM + 1
CLUSTER_NUM = 10000
EMBED_SIZE = 64

_info = plsc.get_sparse_core_info()
_NC, _NS = _info.num_cores, _info.num_subcores
_NW = _NC * _NS

_N = 4096 * 200
_BPW = _N // _NW
_RMAP_PAD = 100096
_C = 320
_NCHUNK = _BPW // _C

_mesh = plsc.VectorSubcoreMesh(core_axis_name="c", subcore_axis_name="s")


def _wid():
    return lax.axis_index("s") * _NC + lax.axis_index("c")


@functools.partial(
    pl.kernel,
    mesh=_mesh,
    compiler_params=pltpu.CompilerParams(use_tc_tiling_on_sc=False),
    out_type=jax.ShapeDtypeStruct((_N,), jnp.int32),
    scratch_types=[
        pltpu.VMEM((_BPW,), jnp.int32),
        pltpu.VMEM((_BPW,), jnp.int32),
        pltpu.SemaphoreType.DMA,
    ],
)
def _level1(idx_hbm, rmap_hbm, cid_hbm, idx_v, cid_v, sem):
    base = _wid() * _BPW
    pltpu.sync_copy(idx_hbm.at[pl.ds(base, _BPW)], idx_v)
    pltpu.async_copy(rmap_hbm.at[idx_v], cid_v, sem).wait()
    pltpu.sync_copy(cid_v, cid_hbm.at[pl.ds(base, _BPW)])


@functools.partial(
    pl.kernel,
    mesh=_mesh,
    out_type=jax.ShapeDtypeStruct((_N, 2 * EMBED_SIZE), jnp.float32),
    scratch_types=[
        pltpu.VMEM((_BPW,), jnp.int32),
        pltpu.VMEM((_C, 2 * EMBED_SIZE), jnp.float32),
        pltpu.VMEM((_C, 2 * EMBED_SIZE), jnp.float32),
        pltpu.SemaphoreType.DMA,
        pltpu.SemaphoreType.DMA,
    ],
)
def _level2(cid_hbm, tbl_hbm, out_hbm, cid_v, rows_a, rows_b, sem_a, sem_b):
    base = _wid() * _BPW
    pltpu.sync_copy(cid_hbm.at[pl.ds(base, _BPW)], cid_v)

    bufs = (rows_a, rows_b)
    sems = (sem_a, sem_b)

    def gather(c, buf, sem):
        return pltpu.async_copy(tbl_hbm.at[cid_v.at[pl.ds(c * _C, _C)]],
                                buf, sem)

    gather(0, bufs[0], sems[0])

    def step(c, _):
        par = lax.rem(c, 2)

        def handle(b):
            @pl.when(par == b)
            def _():
                nxt = c + 1

                @pl.when(nxt < _NCHUNK)
                def _():
                    gather(nxt, bufs[1 - b], sems[1 - b])

                pltpu.make_async_copy(
                    tbl_hbm.at[cid_v.at[pl.ds(0, _C)]],
                    bufs[b], sems[b]).wait()
                pltpu.sync_copy(bufs[b],
                                out_hbm.at[pl.ds(base + c * _C, _C)])

        handle(0)
        handle(1)
        return 0

    lax.fori_loop(0, _NCHUNK, step, 0)


_BB = 16
_RPB = _BB * 200 // 2


def _untile_body(x_ref, o_ref):
    x = x_ref[...]
    left = x[:, :EMBED_SIZE].reshape(_BB, 100, 1, EMBED_SIZE)
    right = x[:, EMBED_SIZE:].reshape(_BB, 100, 1, EMBED_SIZE)
    o_ref[...] = jnp.concatenate([left, right], axis=2).reshape(
        _BB, 200, EMBED_SIZE)


def _untile(x):
    return pl.pallas_call(
        _untile_body,
        out_shape=jax.ShapeDtypeStruct((4096, 200, EMBED_SIZE), jnp.float32),
        grid=(4096 // _BB,),
        in_specs=[pl.BlockSpec((_RPB, 2 * EMBED_SIZE), lambda i: (i, 0))],
        out_specs=pl.BlockSpec((_BB, 200, EMBED_SIZE), lambda i: (i, 0, 0)),
    )(x)


def kernel(data_orig, road_map, cluster_table):
    flat = data_orig.reshape(-1)
    road_map2 = jnp.pad(road_map.at[PAD_ID].set(CLUSTER_NUM),
                        (0, _RMAP_PAD - (ROUTEID_NUM + 2)))
    table2 = jnp.pad(cluster_table, ((0, 1), (0, EMBED_SIZE)))
    cid = _level1(flat, road_map2)
    out = _level2(cid, table2)
    return _untile(out)


# speedup vs baseline: 1.1613x; 1.1613x over previous
import functools

import jax
import jax.numpy as jnp
from jax import lax
from jax.experimental import pallas as pl
from jax.experimental.pallas import tpu as pltpu
from jax.experimental.pallas import tpu_sc as plsc

ROUTEID_NUM = 100000
PAD_ID = ROUTEID_NUM + 1
CLUSTER_NUM = 10000
EMBED_SIZE = 64

_info = plsc.get_sparse_core_info()
_NC, _NS = _info.num_cores, _info.num_subcores
_NW = _NC * _NS

_N = 4096 * 200
_BPW = _N // _NW
_RMAP_PAD = 100096
_C = 320
_NCHUNK = _BPW // _C

_mesh = plsc.VectorSubcoreMesh(core_axis_name="c", subcore_axis_name="s")


def _wid():
    return lax.axis_index("s") * _NC + lax.axis_index("c")


@functools.partial(
    pl.kernel,
    mesh=_mesh,
    compiler_params=pltpu.CompilerParams(use_tc_tiling_on_sc=False),
    out_type=jax.ShapeDtypeStruct((_N,), jnp.int32),
    scratch_types=[
        pltpu.VMEM((_BPW,), jnp.int32),
        pltpu.VMEM((_BPW,), jnp.int32),
        pltpu.SemaphoreType.DMA,
    ],
)
def _level1(idx_hbm, rmap_hbm, cid_hbm, idx_v, cid_v, sem):
    base = _wid() * _BPW
    pltpu.sync_copy(idx_hbm.at[pl.ds(base, _BPW)], idx_v)
    pltpu.async_copy(rmap_hbm.at[idx_v], cid_v, sem).wait()
    pltpu.sync_copy(cid_v, cid_hbm.at[pl.ds(base, _BPW)])


@functools.partial(
    pl.kernel,
    mesh=_mesh,
    out_type=jax.ShapeDtypeStruct((_N, 2 * EMBED_SIZE), jnp.float32),
    scratch_types=[
        pltpu.VMEM((_BPW,), jnp.int32),
        pltpu.VMEM((_C, 2 * EMBED_SIZE), jnp.float32),
        pltpu.VMEM((_C, 2 * EMBED_SIZE), jnp.float32),
        pltpu.SemaphoreType.DMA,
        pltpu.SemaphoreType.DMA,
    ],
)
def _level2(cid_hbm, tbl_hbm, out_hbm, cid_v, rows_a, rows_b, sem_a, sem_b):
    base = _wid() * _BPW
    pltpu.sync_copy(cid_hbm.at[pl.ds(base, _BPW)], cid_v)

    bufs = (rows_a, rows_b)
    sems = (sem_a, sem_b)

    def gather(c, buf, sem):
        return pltpu.async_copy(tbl_hbm.at[cid_v.at[pl.ds(c * _C, _C)]],
                                buf, sem)

    gather(0, bufs[0], sems[0])

    def step(c, _):
        par = lax.rem(c, 2)

        def handle(b):
            @pl.when(par == b)
            def _():
                nxt = c + 1

                @pl.when(nxt < _NCHUNK)
                def _():
                    gather(nxt, bufs[1 - b], sems[1 - b])

                pltpu.make_async_copy(
                    tbl_hbm.at[cid_v.at[pl.ds(0, _C)]],
                    bufs[b], sems[b]).wait()
                pltpu.sync_copy(bufs[b],
                                out_hbm.at[pl.ds(base + c * _C, _C)])

        handle(0)
        handle(1)
        return 0

    lax.fori_loop(0, _NCHUNK, step, 0)


_BB = 16
_RPB = _BB * 200


def _untile_body(x_ref, o_ref):
    o_ref[...] = x_ref[:, :EMBED_SIZE].reshape(_BB, 200, EMBED_SIZE)


def _untile(x):
    return pl.pallas_call(
        _untile_body,
        out_shape=jax.ShapeDtypeStruct((4096, 200, EMBED_SIZE), jnp.float32),
        grid=(4096 // _BB,),
        in_specs=[pl.BlockSpec((_RPB, 2 * EMBED_SIZE), lambda i: (i, 0))],
        out_specs=pl.BlockSpec((_BB, 200, EMBED_SIZE), lambda i: (i, 0, 0)),
    )(x)


def kernel(data_orig, road_map, cluster_table):
    flat = data_orig.reshape(-1)
    road_map2 = jnp.pad(road_map.at[PAD_ID].set(CLUSTER_NUM),
                        (0, _RMAP_PAD - (ROUTEID_NUM + 2)))
    table2 = jnp.pad(cluster_table, ((0, 1), (0, EMBED_SIZE)))
    cid = _level1(flat, road_map2)
    out = _level2(cid, table2)
    return _untile(out)


# speedup vs baseline: 2.6324x; 2.2669x over previous
import functools

import jax
import jax.numpy as jnp
from jax import lax
from jax.experimental import pallas as pl
from jax.experimental.pallas import tpu as pltpu
from jax.experimental.pallas import tpu_sc as plsc

ROUTEID_NUM = 100000
PAD_ID = ROUTEID_NUM + 1
CLUSTER_NUM = 10000
EMBED_SIZE = 64

_info = plsc.get_sparse_core_info()
_NC, _NS = _info.num_cores, _info.num_subcores
_NW = _NC * _NS

_N = 4096 * 200
_BPW = _N // _NW
_RMAP_PAD = 100096
_TROWS = 10240
_SROWS = _TROWS // _NS
_C = 160
_S = 3200
_CPS = _S // _C
_NSUP = _BPW // _S

_mesh = plsc.VectorSubcoreMesh(core_axis_name="c", subcore_axis_name="s")


def _wid():
    return lax.axis_index("s") * _NC + lax.axis_index("c")


@functools.partial(
    pl.kernel,
    mesh=_mesh,
    compiler_params=pltpu.CompilerParams(use_tc_tiling_on_sc=False),
    out_type=jax.ShapeDtypeStruct((_N,), jnp.int32),
    scratch_types=[
        pltpu.VMEM((_BPW,), jnp.int32),
        pltpu.VMEM((_BPW,), jnp.int32),
        pltpu.SemaphoreType.DMA,
    ],
)
def _level1(idx_hbm, rmap_hbm, cid_hbm, idx_v, cid_v, sem):
    base = _wid() * _BPW
    pltpu.sync_copy(idx_hbm.at[pl.ds(base, _BPW)], idx_v)
    pltpu.async_copy(rmap_hbm.at[idx_v], cid_v, sem).wait()
    pltpu.sync_copy(cid_v, cid_hbm.at[pl.ds(base, _BPW)])


@functools.partial(
    pl.kernel,
    mesh=_mesh,
    out_type=jax.ShapeDtypeStruct((_N, 2 * EMBED_SIZE), jnp.float32),
    scratch_types=[
        pltpu.VMEM((_S,), jnp.int32),
        pltpu.VMEM((_S,), jnp.int32),
        pltpu.VMEM((_C, 2 * EMBED_SIZE), jnp.float32),
        pltpu.VMEM((_C, 2 * EMBED_SIZE), jnp.float32),
        pltpu.VMEM_SHARED((_TROWS, 2 * EMBED_SIZE), jnp.float32),
        pltpu.SemaphoreType.DMA,
        pltpu.SemaphoreType.DMA,
        pltpu.SemaphoreType.DMA,
        pltpu.SemaphoreType.DMA,
    ],
)
def _level2(cid_hbm, tbl_hbm, out_hbm,
            cid_a, cid_b, rows_a, rows_b, spm,
            sem_ca, sem_cb, sem_a, sem_b):
    sid = lax.axis_index("s")
    base = _wid() * _BPW

    for t in range(_SROWS // _C):
        off = sid * _SROWS + t * _C
        pltpu.sync_copy(tbl_hbm.at[pl.ds(off, _C)], rows_a)
        pltpu.sync_copy(rows_a, spm.at[pl.ds(off, _C)])
    plsc.subcore_barrier()

    cbufs = (cid_a, cid_b)
    csems = (sem_ca, sem_cb)
    bufs = (rows_a, rows_b)
    sems = (sem_a, sem_b)

    def cid_fetch(s):
        return pltpu.async_copy(cid_hbm.at[pl.ds(base + s * _S, _S)],
                                cbufs[s % 2], csems[s % 2])

    def cid_wait(s):
        pltpu.make_async_copy(cid_hbm.at[pl.ds(base, _S)],
                              cbufs[s % 2], csems[s % 2]).wait()

    def gather(s, k, b):
        return pltpu.async_copy(
            spm.at[cbufs[s % 2].at[pl.ds(k * _C, _C)]], bufs[b], sems[b])

    def drain_write(s, k, b):
        pltpu.make_async_copy(spm.at[cbufs[0].at[pl.ds(0, _C)]],
                              bufs[b], sems[b]).wait()
        pltpu.sync_copy(bufs[b],
                        out_hbm.at[pl.ds(base + (s * _CPS + k) * _C, _C)])

    cid_fetch(0)
    cid_wait(0)
    cid_fetch(1)
    gather(0, 0, 0)

    for s in range(_NSUP):
        def step(k, _):
            par = lax.rem(k, 2)

            def handle(b):
                @pl.when(par == b)
                def _():
                    @pl.when(k < _CPS - 1)
                    def _():
                        gather(s, k + 1, 1 - b)
                    drain_write(s, k, b)

            handle(0)
            handle(1)
            return 0

        lax.fori_loop(0, _CPS, step, 0)

        if s + 1 < _NSUP:
            cid_wait(s + 1)
            if s + 2 < _NSUP:
                cid_fetch(s + 2)
            gather(s + 1, 0, 0)


def kernel(data_orig, road_map, cluster_table):
    flat = data_orig.reshape(-1)
    road_map2 = jnp.pad(road_map.at[PAD_ID].set(CLUSTER_NUM),
                        (0, _RMAP_PAD - (ROUTEID_NUM + 2)))
    table2 = jnp.pad(cluster_table,
                     ((0, _TROWS - CLUSTER_NUM), (0, EMBED_SIZE)))
    cid = _level1(flat, road_map2)
    out = _level2(cid, table2)
    out = out[:, :EMBED_SIZE]
    return out.reshape(data_orig.shape[0], data_orig.shape[1], EMBED_SIZE)


# speedup vs baseline: 2.7915x; 1.0604x over previous
import functools

import jax
import jax.numpy as jnp
from jax import lax
from jax.experimental import pallas as pl
from jax.experimental.pallas import tpu as pltpu
from jax.experimental.pallas import tpu_sc as plsc

ROUTEID_NUM = 100000
PAD_ID = ROUTEID_NUM + 1
CLUSTER_NUM = 10000
EMBED_SIZE = 64

_info = plsc.get_sparse_core_info()
_NC, _NS = _info.num_cores, _info.num_subcores
_NW = _NC * _NS

_N = 4096 * 200
_BPW = _N // _NW
_RMAP_PAD = 100096
_TROWS = 10240
_SROWS = _TROWS // _NS
_C = 160
_S = 1600
_CPS = _S // _C
_NSUP = _BPW // _S

_mesh = plsc.VectorSubcoreMesh(core_axis_name="c", subcore_axis_name="s")


@functools.partial(
    pl.kernel,
    mesh=_mesh,
    out_type=jax.ShapeDtypeStruct((_N, 2 * EMBED_SIZE), jnp.float32),
    scratch_types=[
        pltpu.VMEM((_S,), jnp.int32),
        pltpu.VMEM((_S,), jnp.int32),
        pltpu.VMEM((_S,), jnp.int32),
        pltpu.VMEM((_S,), jnp.int32),
        pltpu.VMEM((_C, 2 * EMBED_SIZE), jnp.float32),
        pltpu.VMEM((_C, 2 * EMBED_SIZE), jnp.float32),
        pltpu.VMEM_SHARED((_TROWS, 2 * EMBED_SIZE), jnp.float32),
        pltpu.SemaphoreType.DMA,
        pltpu.SemaphoreType.DMA,
        pltpu.SemaphoreType.DMA,
        pltpu.SemaphoreType.DMA,
        pltpu.SemaphoreType.DMA,
        pltpu.SemaphoreType.DMA,
    ],
)
def _lookup(idx_hbm, rmap_hbm, tbl_hbm, out_hbm,
            idx_a, idx_b, cid_a, cid_b, rows_a, rows_b, spm,
            sem_ia, sem_ib, sem_ca, sem_cb, sem_a, sem_b):
    sid = lax.axis_index("s")
    base = (sid * _NC + lax.axis_index("c")) * _BPW

    ibufs, isems = (idx_a, idx_b), (sem_ia, sem_ib)
    cbufs, csems = (cid_a, cid_b), (sem_ca, sem_cb)
    bufs, sems = (rows_a, rows_b), (sem_a, sem_b)

    def idx_fetch(s):
        return pltpu.async_copy(idx_hbm.at[pl.ds(base + s * _S, _S)],
                                ibufs[s % 2], isems[s % 2])

    def idx_wait(s):
        pltpu.make_async_copy(idx_hbm.at[pl.ds(base, _S)],
                              ibufs[s % 2], isems[s % 2]).wait()

    def cid_gather(s):
        return pltpu.async_copy(rmap_hbm.at[ibufs[s % 2]],
                                cbufs[s % 2], csems[s % 2])

    def cid_wait(s):
        pltpu.make_async_copy(rmap_hbm.at[pl.ds(0, _S)],
                              cbufs[s % 2], csems[s % 2]).wait()

    def gather(s, k, b):
        return pltpu.async_copy(
            spm.at[cbufs[s % 2].at[pl.ds(k * _C, _C)]], bufs[b], sems[b])

    def drain_write(s, k, b):
        pltpu.make_async_copy(spm.at[cbufs[0].at[pl.ds(0, _C)]],
                              bufs[b], sems[b]).wait()
        pltpu.sync_copy(bufs[b],
                        out_hbm.at[pl.ds(base + (s * _CPS + k) * _C, _C)])

    idx_fetch(0)
    idx_fetch(1)
    for t in range(_SROWS // _C):
        off = sid * _SROWS + t * _C
        pltpu.sync_copy(tbl_hbm.at[pl.ds(off, _C)], rows_a)
        pltpu.sync_copy(rows_a, spm.at[pl.ds(off, _C)])
    plsc.subcore_barrier()
    idx_wait(0)
    cid_gather(0)
    cid_wait(0)
    gather(0, 0, 0)

    for s in range(_NSUP):
        if s + 1 < _NSUP:
            idx_wait(s + 1)
            cid_gather(s + 1)
            if s + 2 < _NSUP:
                idx_fetch(s + 2)

        def step(k, _):
            par = lax.rem(k, 2)

            def handle(b):
                @pl.when(par == b)
                def _():
                    @pl.when(k < _CPS - 1)
                    def _():
                        gather(s, k + 1, 1 - b)
                    drain_write(s, k, b)

            handle(0)
            handle(1)
            return 0

        lax.fori_loop(0, _CPS, step, 0)

        if s + 1 < _NSUP:
            cid_wait(s + 1)
            gather(s + 1, 0, 0)


def kernel(data_orig, road_map, cluster_table):
    flat = data_orig.reshape(-1)
    road_map2 = jnp.pad(road_map.at[PAD_ID].set(CLUSTER_NUM),
                        (0, _RMAP_PAD - (ROUTEID_NUM + 2)))
    table2 = jnp.pad(cluster_table,
                     ((0, _TROWS - CLUSTER_NUM), (0, EMBED_SIZE)))
    out = _lookup(flat, road_map2, table2)
    out = out[:, :EMBED_SIZE]
    return out.reshape(data_orig.shape[0], data_orig.shape[1], EMBED_SIZE)


# speedup vs baseline: 2.8900x; 1.0353x over previous
import functools

import jax
import jax.numpy as jnp
from jax import lax
from jax.experimental import pallas as pl
from jax.experimental.pallas import tpu as pltpu
from jax.experimental.pallas import tpu_sc as plsc

ROUTEID_NUM = 100000
PAD_ID = ROUTEID_NUM + 1
CLUSTER_NUM = 10000
EMBED_SIZE = 64

_info = plsc.get_sparse_core_info()
_NC, _NS = _info.num_cores, _info.num_subcores
_NW = _NC * _NS

_N = 4096 * 200
_BPW = _N // _NW
_RMAP_PAD = 100096
_TROWS = 10240
_SROWS = _TROWS // _NS
_C = 160
_S = 1600
_CPS = _S // _C
_NSUP = _BPW // _S

_mesh = plsc.VectorSubcoreMesh(core_axis_name="c", subcore_axis_name="s")


@functools.partial(
    pl.kernel,
    mesh=_mesh,
    out_type=jax.ShapeDtypeStruct((_N, 2 * EMBED_SIZE), jnp.float32),
    scratch_types=[
        pltpu.VMEM((_S,), jnp.int32),
        pltpu.VMEM((_S,), jnp.int32),
        pltpu.VMEM((_S,), jnp.int32),
        pltpu.VMEM((_S,), jnp.int32),
        pltpu.VMEM((_C, 2 * EMBED_SIZE), jnp.float32),
        pltpu.VMEM((_C, 2 * EMBED_SIZE), jnp.float32),
        pltpu.VMEM_SHARED((_TROWS, 2 * EMBED_SIZE), jnp.float32),
        pltpu.SemaphoreType.DMA,
        pltpu.SemaphoreType.DMA,
        pltpu.SemaphoreType.DMA,
        pltpu.SemaphoreType.DMA,
        pltpu.SemaphoreType.DMA,
        pltpu.SemaphoreType.DMA,
        pltpu.SemaphoreType.DMA,
        pltpu.SemaphoreType.DMA,
    ],
)
def _lookup(idx_hbm, rmap_hbm, tbl_hbm, out_hbm,
            idx_a, idx_b, cid_a, cid_b, rows_a, rows_b, spm,
            sem_ia, sem_ib, sem_ca, sem_cb, sem_a, sem_b, sem_wa, sem_wb):
    sid = lax.axis_index("s")
    base = (sid * _NC + lax.axis_index("c")) * _BPW

    ibufs, isems = (idx_a, idx_b), (sem_ia, sem_ib)
    cbufs, csems = (cid_a, cid_b), (sem_ca, sem_cb)
    bufs, sems = (rows_a, rows_b), (sem_a, sem_b)
    wsems = (sem_wa, sem_wb)

    def idx_fetch(s):
        return pltpu.async_copy(idx_hbm.at[pl.ds(base + s * _S, _S)],
                                ibufs[s % 2], isems[s % 2])

    def idx_wait(s):
        pltpu.make_async_copy(idx_hbm.at[pl.ds(base, _S)],
                              ibufs[s % 2], isems[s % 2]).wait()

    def cid_gather(s):
        return pltpu.async_copy(rmap_hbm.at[ibufs[s % 2]],
                                cbufs[s % 2], csems[s % 2])

    def cid_wait(s):
        pltpu.make_async_copy(rmap_hbm.at[pl.ds(0, _S)],
                              cbufs[s % 2], csems[s % 2]).wait()

    def gather(s, k, b):
        return pltpu.async_copy(
            spm.at[cbufs[s % 2].at[pl.ds(k * _C, _C)]], bufs[b], sems[b])

    def drain_write(s, k, b):
        pltpu.make_async_copy(spm.at[cbufs[0].at[pl.ds(0, _C)]],
                              bufs[b], sems[b]).wait()
        pltpu.async_copy(bufs[b],
                         out_hbm.at[pl.ds(base + (s * _CPS + k) * _C, _C)],
                         wsems[b])

    def write_wait(b):
        pltpu.make_async_copy(bufs[b], out_hbm.at[pl.ds(base, _C)],
                              wsems[b]).wait()

    idx_fetch(0)
    idx_fetch(1)
    for t in range(_SROWS // _C):
        off = sid * _SROWS + t * _C
        pltpu.sync_copy(tbl_hbm.at[pl.ds(off, _C)], rows_a)
        pltpu.sync_copy(rows_a, spm.at[pl.ds(off, _C)])
    plsc.subcore_barrier()
    idx_wait(0)
    cid_gather(0)
    cid_wait(0)
    gather(0, 0, 0)

    for s in range(_NSUP):
        if s + 1 < _NSUP:
            idx_wait(s + 1)
            cid_gather(s + 1)
            if s + 2 < _NSUP:
                idx_fetch(s + 2)

        def step(k, _):
            par = lax.rem(k, 2)

            def handle(b):
                @pl.when(par == b)
                def _():
                    @pl.when(k + s > 0)
                    def _():
                        write_wait(1 - b)

                    @pl.when(k < _CPS - 1)
                    def _():
                        gather(s, k + 1, 1 - b)
                    drain_write(s, k, b)

            handle(0)
            handle(1)
            return 0

        lax.fori_loop(0, _CPS, step, 0)

        if s + 1 < _NSUP:
            cid_wait(s + 1)
            gather(s + 1, 0, 0)
    write_wait(1)


def kernel(data_orig, road_map, cluster_table):
    flat = data_orig.reshape(-1)
    road_map2 = jnp.pad(road_map.at[PAD_ID].set(CLUSTER_NUM),
                        (0, _RMAP_PAD - (ROUTEID_NUM + 2)))
    table2 = jnp.pad(cluster_table,
                     ((0, _TROWS - CLUSTER_NUM), (0, EMBED_SIZE)))
    out = _lookup(flat, road_map2, table2)
    out = out[:, :EMBED_SIZE]
    return out.reshape(data_orig.shape[0], data_orig.shape[1], EMBED_SIZE)


# speedup vs baseline: 2.8911x; 1.0004x over previous
import functools

import jax
import jax.numpy as jnp
from jax import lax
from jax.experimental import pallas as pl
from jax.experimental.pallas import tpu as pltpu
from jax.experimental.pallas import tpu_sc as plsc

ROUTEID_NUM = 100000
PAD_ID = ROUTEID_NUM + 1
CLUSTER_NUM = 10000
EMBED_SIZE = 64

_info = plsc.get_sparse_core_info()
_NC, _NS = _info.num_cores, _info.num_subcores
_NW = _NC * _NS

_N = 4096 * 200
_BPW = _N // _NW
_RMAP_PAD = 100096
_TROWS = 10240
_SROWS = _TROWS // _NS
_C = 160
_S = 1600
_CPS = _S // _C
_NSUP = _BPW // _S

_mesh = plsc.VectorSubcoreMesh(core_axis_name="c", subcore_axis_name="s")


@functools.partial(
    pl.kernel,
    mesh=_mesh,
    out_type=jax.ShapeDtypeStruct((_N, 2 * EMBED_SIZE), jnp.float32),
    scratch_types=[
        pltpu.VMEM((_S,), jnp.int32),
        pltpu.VMEM((_S,), jnp.int32),
        pltpu.VMEM((_S,), jnp.int32),
        pltpu.VMEM((_S,), jnp.int32),
        pltpu.VMEM((_C, 2 * EMBED_SIZE), jnp.float32),
        pltpu.VMEM((_C, 2 * EMBED_SIZE), jnp.float32),
        pltpu.VMEM_SHARED((_TROWS, 2 * EMBED_SIZE), jnp.float32),
        pltpu.SemaphoreType.DMA,
        pltpu.SemaphoreType.DMA,
        pltpu.SemaphoreType.DMA,
        pltpu.SemaphoreType.DMA,
        pltpu.SemaphoreType.DMA,
        pltpu.SemaphoreType.DMA,
        pltpu.SemaphoreType.DMA,
        pltpu.SemaphoreType.DMA,
    ],
)
def _lookup(idx_hbm, rmap_hbm, tbl_hbm, out_hbm,
            idx_a, idx_b, cid_a, cid_b, rows_a, rows_b, spm,
            sem_ia, sem_ib, sem_ca, sem_cb, sem_a, sem_b, sem_wa, sem_wb):
    sid = lax.axis_index("s")
    base = (sid * _NC + lax.axis_index("c")) * _BPW

    ibufs, isems = (idx_a, idx_b), (sem_ia, sem_ib)
    cbufs, csems = (cid_a, cid_b), (sem_ca, sem_cb)
    bufs, sems = (rows_a, rows_b), (sem_a, sem_b)
    wsems = (sem_wa, sem_wb)

    def idx_fetch(s):
        return pltpu.async_copy(idx_hbm.at[pl.ds(base + s * _S, _S)],
                                ibufs[s % 2], isems[s % 2])

    def idx_wait(s):
        pltpu.make_async_copy(idx_hbm.at[pl.ds(base, _S)],
                              ibufs[s % 2], isems[s % 2]).wait()

    def cid_gather(s):
        return pltpu.async_copy(rmap_hbm.at[ibufs[s % 2]],
                                cbufs[s % 2], csems[s % 2])

    def cid_wait(s):
        pltpu.make_async_copy(rmap_hbm.at[pl.ds(0, _S)],
                              cbufs[s % 2], csems[s % 2]).wait()

    def gather(s, k, b):
        return pltpu.async_copy(
            spm.at[cbufs[s % 2].at[pl.ds(k * _C, _C)]], bufs[b], sems[b])

    def drain_write(s, k, b):
        pltpu.make_async_copy(spm.at[cbufs[0].at[pl.ds(0, _C)]],
                              bufs[b], sems[b]).wait()
        pltpu.async_copy(bufs[b],
                         out_hbm.at[pl.ds(base + (s * _CPS + k) * _C, _C)],
                         wsems[b])

    def write_wait(b):
        pltpu.make_async_copy(bufs[b], out_hbm.at[pl.ds(base, _C)],
                              wsems[b]).wait()

    idx_fetch(0)
    idx_fetch(1)
    for t in range(_SROWS // _C):
        off = sid * _SROWS + t * _C
        pltpu.sync_copy(tbl_hbm.at[pl.ds(off, _C)], rows_a)
        pltpu.sync_copy(rows_a, spm.at[pl.ds(off, _C)])
    plsc.subcore_barrier()
    idx_wait(0)
    cid_gather(0)
    cid_wait(0)
    gather(0, 0, 0)

    for s in range(_NSUP):
        if s + 1 < _NSUP:
            idx_wait(s + 1)
            cid_gather(s + 1)
            if s + 2 < _NSUP:
                idx_fetch(s + 2)

        def step(k, _):
            par = lax.rem(k, 2)

            def handle(b):
                @pl.when(par == b)
                def _():
                    @pl.when(k + s > 0)
                    def _():
                        write_wait(1 - b)

                    @pl.when(k < _CPS - 1)
                    def _():
                        gather(s, k + 1, 1 - b)
                    if s + 1 < _NSUP:
                        @pl.when(k == _CPS - 1)
                        def _():
                            cid_wait(s + 1)
                            gather(s + 1, 0, 1 - b)
                    drain_write(s, k, b)

            handle(0)
            handle(1)
            return 0

        lax.fori_loop(0, _CPS, step, 0)
    write_wait(1)


def kernel(data_orig, road_map, cluster_table):
    flat = data_orig.reshape(-1)
    road_map2 = jnp.pad(road_map.at[PAD_ID].set(CLUSTER_NUM),
                        (0, _RMAP_PAD - (ROUTEID_NUM + 2)))
    table2 = jnp.pad(cluster_table,
                     ((0, _TROWS - CLUSTER_NUM), (0, EMBED_SIZE)))
    out = _lookup(flat, road_map2, table2)
    out = out[:, :EMBED_SIZE]
    return out.reshape(data_orig.shape[0], data_orig.shape[1], EMBED_SIZE)
